# baseline (speedup 1.0000x reference)
import functools

import jax
import jax.numpy as jnp
from jax import lax
from jax.experimental import pallas as pl
from jax.experimental.pallas import tpu as pltpu

_EPS = 1e-5


def _encode_kernel(L, tok_ref, emb_ref, s1_ref, t1_ref, w1_ref, b1_ref,
                   s2_ref, t2_ref, g_ref, a_scr):
    Bblk = g_ref.shape[0]
    b0 = pl.program_id(0) * Bblk
    D = emb_ref.shape[1]
    sub_iota = lax.broadcasted_iota(jnp.int32, (8, D), 0)

    def group(gi, carry):
        e8 = jnp.zeros((8, D), jnp.float32)
        for j in range(8):
            base = (b0 + gi * 8 + j) * L
            accs = [None, None]
            for l in range(L):
                t = tok_ref[base + l]
                cbase = pl.multiple_of((t >> 3) << 3, 8)
                chunk = emb_ref[pl.ds(cbase, 8), :]
                r = pltpu.roll(chunk, -(t & 7), axis=0)
                c = l & 1
                accs[c] = r if accs[c] is None else accs[c] + r
            acc = accs[0] if accs[1] is None else accs[0] + accs[1]
            e8 = e8 + jnp.where(sub_iota == j, pltpu.roll(acc, j, axis=0), 0.0)
        a8 = jnp.maximum(e8, 0.0) * s1_ref[...] + t1_ref[...]
        a_scr[pl.ds(gi * 8, 8), :] = a8
        return carry

    lax.fori_loop(0, Bblk // 8, group, 0)

    h = jnp.dot(a_scr[...], w1_ref[...], preferred_element_type=jnp.float32)
    h = jnp.maximum(h + b1_ref[...], 0.0)
    g_ref[...] = (h * s2_ref[...] + t2_ref[...]).astype(jnp.bfloat16)


def _out_kernel(g_ref, w2_ref, b2_ref, o_ref):
    w = w2_ref[...].astype(jnp.bfloat16)
    o_ref[...] = (
        jnp.dot(g_ref[...], w, preferred_element_type=jnp.float32) + b2_ref[...]
    )


def kernel(tokens, emb, bn1_gamma, bn1_beta, bn1_mean, bn1_var, w1, b1,
           bn2_gamma, bn2_beta, bn2_mean, bn2_var, w2, b2):
    B, L = tokens.shape
    V, D = emb.shape
    Dh = w1.shape[1]
    N = w2.shape[1]

    s1 = bn1_gamma * lax.rsqrt(bn1_var + _EPS)
    t1 = bn1_beta - bn1_mean * s1
    s2 = bn2_gamma * lax.rsqrt(bn2_var + _EPS)
    t2 = bn2_beta - bn2_mean * s2

    tokens_flat = tokens.reshape(-1).astype(jnp.int32)

    g = pl.pallas_call(
        functools.partial(_encode_kernel, L),
        out_shape=jax.ShapeDtypeStruct((B, Dh), jnp.bfloat16),
        grid=(2,),
        in_specs=[
            pl.BlockSpec(memory_space=pltpu.MemorySpace.SMEM),
            pl.BlockSpec((V, D), lambda j: (0, 0)),
            pl.BlockSpec((1, D), lambda j: (0, 0)),
            pl.BlockSpec((1, D), lambda j: (0, 0)),
            pl.BlockSpec((D, Dh), lambda j: (0, 0)),
            pl.BlockSpec((1, Dh), lambda j: (0, 0)),
            pl.BlockSpec((1, Dh), lambda j: (0, 0)),
            pl.BlockSpec((1, Dh), lambda j: (0, 0)),
        ],
        out_specs=pl.BlockSpec((B // 2, Dh), lambda j: (j, 0)),
        scratch_shapes=[pltpu.VMEM((B // 2, D), jnp.float32)],
        compiler_params=pltpu.CompilerParams(
            dimension_semantics=("parallel",),
            vmem_limit_bytes=60 * 1024 * 1024,
        ),
    )(tokens_flat, emb, s1, t1, w1, b1, s2, t2)

    tn = 512 if N % 512 == 0 else N
    out = pl.pallas_call(
        _out_kernel,
        out_shape=jax.ShapeDtypeStruct((B, N), jnp.float32),
        grid=(N // tn,),
        in_specs=[
            pl.BlockSpec((B, Dh), lambda j: (0, 0)),
            pl.BlockSpec((Dh, tn), lambda j: (0, j)),
            pl.BlockSpec((1, tn), lambda j: (0, j)),
        ],
        out_specs=pl.BlockSpec((B, tn), lambda j: (0, j)),
        compiler_params=pltpu.CompilerParams(
            dimension_semantics=("parallel",),
            vmem_limit_bytes=32 * 1024 * 1024,
        ),
    )(g, w2, b2)
    return out
